# baseline (speedup 1.0000x reference)
import functools

import jax
import jax.numpy as jnp
from jax import lax
from jax.experimental import pallas as pl
from jax.experimental.pallas import tpu as pltpu
from jax.experimental.pallas import tpu_sc as plsc

EMB = 128
ENC = 128
NUM_NODES = 15
CHUNK = 128


@functools.lru_cache(maxsize=None)
def _make_sc_gather(vocab: int, rows_total: int):
    info = plsc.get_sparse_core_info()
    nw = info.num_cores * info.num_subcores
    assert rows_total % (nw * CHUNK) == 0
    rows_per_w = rows_total // nw
    n_chunks = rows_per_w // CHUNK
    mesh = plsc.VectorSubcoreMesh(core_axis_name="c", subcore_axis_name="s")

    @functools.partial(
        pl.kernel,
        out_type=jax.ShapeDtypeStruct((rows_total, EMB), jnp.float32),
        mesh=mesh,
        scratch_types=[
            pltpu.VMEM((rows_per_w,), jnp.int32),
            pltpu.VMEM((2, CHUNK, EMB), jnp.float32),
            pltpu.SemaphoreType.DMA,
            pltpu.SemaphoreType.DMA,
        ],
    )
    def gather_kernel(table_hbm, idx_hbm, out_hbm, idx_v, rows_v, gsem, osem):
        wid = lax.axis_index("s") * info.num_cores + lax.axis_index("c")
        base = wid * rows_per_w
        pltpu.sync_copy(idx_hbm.at[pl.ds(base, rows_per_w)], idx_v)

        def gather_start(t, buf):
            return pltpu.async_copy(
                table_hbm.at[idx_v.at[pl.ds(t * CHUNK, CHUNK)]],
                rows_v.at[buf], gsem)

        gather_start(0, 0).wait()
        def step(t, _):
            buf = lax.rem(t, 2)

            @pl.when(t + 1 < n_chunks)
            def _():
                gather_start(t + 1, 1 - buf)

            cp = pltpu.async_copy(
                rows_v.at[buf], out_hbm.at[pl.ds(base + t * CHUNK, CHUNK)],
                osem)

            @pl.when(t + 1 < n_chunks)
            def _():
                pltpu.make_async_copy(
                    table_hbm.at[idx_v.at[pl.ds(0, CHUNK)]],
                    rows_v.at[1 - buf], gsem).wait()
            cp.wait()
            return 0

        lax.fori_loop(0, n_chunks, step, 0)

    return gather_kernel


def _tree_body(x_ref, wih_ref, whh_ref, bleaf_ref, brz_ref, bin_ref, bhn_ref,
               sw_ref, sb_ref, cw_ref, out_ref):
    bf16 = jnp.bfloat16
    wih = wih_ref[...].astype(bf16)
    whh = whh_ref[...].astype(bf16)
    bleaf = bleaf_ref[...]
    brz = brz_ref[...]
    b_in = bin_ref[...]
    b_hn = bhn_ref[...]
    sw = sw_ref[...].astype(bf16)
    sb = sb_ref[...]
    cw = cw_ref[...]

    def mm_t(a, b):
        return lax.dot_general(a.astype(bf16), b, (((1,), (1,)), ((), ())),
                               preferred_element_type=jnp.float32)

    def mm(a, b):
        return lax.dot_general(a.astype(bf16), b, (((1,), (0,)), ((), ())),
                               preferred_element_type=jnp.float32)

    def stanh(x):
        return jnp.tanh(0.5 * x)

    def gru_leaf(xn):
        gi = mm_t(xn, wih) + bleaf
        r = jax.nn.sigmoid(gi[:, :ENC])
        z = jax.nn.sigmoid(gi[:, ENC:2 * ENC])
        n = jnp.tanh(gi[:, 2 * ENC:] + r * b_hn)
        return n - z * n

    def gru_internal(xn, h):
        gi = mm_t(xn, wih)
        gh = mm_t(h, whh)
        grz = gi[:, :2 * ENC] + gh[:, :2 * ENC] + brz
        r = jax.nn.sigmoid(grz[:, :ENC])
        z = jax.nn.sigmoid(grz[:, ENC:])
        h_n = gh[:, 2 * ENC:] + b_hn
        n = jnp.tanh(gi[:, 2 * ENC:] + b_in + r * h_n)
        return n + z * (h - n)

    def level_attn(hls, hrs):
        def u_of(h):
            w1 = jnp.tanh(mm(h, sw) + sb)
            return jnp.sum(w1 * cw, axis=1, keepdims=True)
        k = len(hls)
        u = jnp.concatenate([u_of(h) for h in hls + hrs], axis=1)
        t = jnp.tanh(u)
        d = t[:, :k] - t[:, k:]
        a = 0.5 * stanh(d) + 0.5
        return [hrs[i] + a[:, i:i + 1] * (hls[i] - hrs[i]) for i in range(k)]

    hs = [None] * NUM_NODES
    for n in range(NUM_NODES - 1, 6, -1):
        hs[n] = gru_leaf(x_ref[n])
    for lo, hi in ((3, 7), (1, 3), (0, 1)):
        parents = list(range(lo, hi))
        hsums = level_attn([hs[2 * n + 1] for n in parents],
                           [hs[2 * n + 2] for n in parents])
        for n, hsum in zip(parents, hsums):
            hs[n] = gru_internal(x_ref[n], hsum)

    m = hs[0]
    for n in range(1, NUM_NODES):
        m = jnp.maximum(m, hs[n])
    out_ref[...] = m


def _tree_encode(x, W_ih, W_hh, b_ih, b_hh, sent_weight, sent_bias, cw_row,
                 block_b: int):
    batch = x.shape[1]
    nb = batch // block_b
    b_leaf = (b_ih + jnp.concatenate(
        [b_hh[:, :2 * ENC], jnp.zeros((1, ENC), jnp.float32)], axis=1))
    b_rz = b_ih[:, :2 * ENC] + b_hh[:, :2 * ENC]
    b_in = b_ih[:, 2 * ENC:]
    b_hn = b_hh[:, 2 * ENC:]
    full = lambda shape: pl.BlockSpec(shape, lambda i: (0,) * len(shape))
    return pl.pallas_call(
        _tree_body,
        grid=(nb,),
        in_specs=[
            pl.BlockSpec((NUM_NODES, block_b, EMB), lambda i: (0, i, 0)),
            full((3 * ENC, EMB)),
            full((3 * ENC, ENC)),
            full((1, 3 * ENC)),
            full((1, 2 * ENC)),
            full((1, ENC)),
            full((1, ENC)),
            full((ENC, ENC)),
            full((1, ENC)),
            full((1, ENC)),
        ],
        out_specs=pl.BlockSpec((block_b, ENC), lambda i: (i, 0)),
        out_shape=jax.ShapeDtypeStruct((batch, ENC), jnp.float32),
        compiler_params=pltpu.CompilerParams(
            dimension_semantics=("parallel",)),
    )(x, W_ih, W_hh, b_leaf, b_rz, b_in, b_hn, sent_weight, sent_bias, cw_row)


NUM_SLICES = 4


def kernel(tokens, bs, emb_table, sent_weight, sent_bias, context_weight,
           W_ih, W_hh, b_ih, b_hh):
    num_nodes, batch = tokens.shape
    bsl = batch // NUM_SLICES
    gather = _make_sc_gather(emb_table.shape[0], num_nodes * bsl)
    outs = []
    for s in range(NUM_SLICES):
        idx = tokens[:, s * bsl:(s + 1) * bsl].reshape(-1).astype(jnp.int32)
        gath = gather(emb_table, idx)
        x = gath.reshape(num_nodes, bsl, EMB)
        outs.append(_tree_encode(
            x, W_ih, W_hh,
            b_ih.reshape(1, -1), b_hh.reshape(1, -1),
            sent_weight, sent_bias, context_weight.reshape(1, -1),
            block_b=1024))
    return jnp.concatenate(outs, axis=0)


# speedup vs baseline: 1.0264x; 1.0264x over previous
import functools

import jax
import jax.numpy as jnp
from jax import lax
from jax.experimental import pallas as pl
from jax.experimental.pallas import tpu as pltpu
from jax.experimental.pallas import tpu_sc as plsc

EMB = 128
ENC = 128
NUM_NODES = 15
CHUNK = 128


@functools.lru_cache(maxsize=None)
def _make_sc_gather(vocab: int, rows_total: int):
    info = plsc.get_sparse_core_info()
    nw = info.num_cores * info.num_subcores
    assert rows_total % (nw * CHUNK) == 0
    rows_per_w = rows_total // nw
    n_chunks = rows_per_w // CHUNK
    mesh = plsc.VectorSubcoreMesh(core_axis_name="c", subcore_axis_name="s")

    @functools.partial(
        pl.kernel,
        out_type=jax.ShapeDtypeStruct((rows_total, EMB), jnp.float32),
        mesh=mesh,
        scratch_types=[
            pltpu.VMEM((rows_per_w,), jnp.int32),
            pltpu.VMEM((2, CHUNK, EMB), jnp.float32),
            pltpu.SemaphoreType.DMA,
            pltpu.SemaphoreType.DMA,
        ],
    )
    def gather_kernel(table_hbm, idx_hbm, out_hbm, idx_v, rows_v, gsem, osem):
        wid = lax.axis_index("s") * info.num_cores + lax.axis_index("c")
        base = wid * rows_per_w
        pltpu.sync_copy(idx_hbm.at[pl.ds(base, rows_per_w)], idx_v)

        def gather_start(t, buf):
            return pltpu.async_copy(
                table_hbm.at[idx_v.at[pl.ds(t * CHUNK, CHUNK)]],
                rows_v.at[buf], gsem)

        gather_start(0, 0).wait()
        def step(t, _):
            buf = lax.rem(t, 2)

            @pl.when(t + 1 < n_chunks)
            def _():
                gather_start(t + 1, 1 - buf)

            cp = pltpu.async_copy(
                rows_v.at[buf], out_hbm.at[pl.ds(base + t * CHUNK, CHUNK)],
                osem)

            @pl.when(t + 1 < n_chunks)
            def _():
                pltpu.make_async_copy(
                    table_hbm.at[idx_v.at[pl.ds(0, CHUNK)]],
                    rows_v.at[1 - buf], gsem).wait()
            cp.wait()
            return 0

        lax.fori_loop(0, n_chunks, step, 0)

    return gather_kernel


def _tree_body(x_ref, wih_ref, whh_ref, bleaf_ref, brz_ref, bin_ref, bhn_ref,
               sw_ref, sb_ref, cw_ref, out_ref):
    bf16 = jnp.bfloat16
    wih = wih_ref[...].astype(bf16)
    whh = whh_ref[...].astype(bf16)
    bleaf = bleaf_ref[...]
    brz = brz_ref[...]
    b_in = bin_ref[...]
    b_hn = bhn_ref[...]
    sw = sw_ref[...].astype(bf16)
    sb = sb_ref[...]
    cw = cw_ref[...]

    def mm_t(a, b):
        return lax.dot_general(a.astype(bf16), b, (((1,), (1,)), ((), ())),
                               preferred_element_type=jnp.float32)

    def mm(a, b):
        return lax.dot_general(a.astype(bf16), b, (((1,), (0,)), ((), ())),
                               preferred_element_type=jnp.float32)

    def stanh(x):
        return jnp.tanh(0.5 * x)

    def gru_leaf(xn):
        gi = mm_t(xn, wih) + bleaf
        r = jax.nn.sigmoid(gi[:, :ENC])
        z = jax.nn.sigmoid(gi[:, ENC:2 * ENC])
        n = jnp.tanh(gi[:, 2 * ENC:] + r * b_hn)
        return n - z * n

    def gru_internal(xn, h):
        gi = mm_t(xn, wih)
        gh = mm_t(h, whh)
        grz = gi[:, :2 * ENC] + gh[:, :2 * ENC] + brz
        r = jax.nn.sigmoid(grz[:, :ENC])
        z = jax.nn.sigmoid(grz[:, ENC:])
        h_n = gh[:, 2 * ENC:] + b_hn
        n = jnp.tanh(gi[:, 2 * ENC:] + b_in + r * h_n)
        return n + z * (h - n)

    def level_attn(hls, hrs):
        def u_of(h):
            w1 = jnp.tanh(mm(h, sw) + sb)
            return jnp.sum(w1 * cw, axis=1, keepdims=True)
        k = len(hls)
        u = jnp.concatenate([u_of(h) for h in hls + hrs], axis=1)
        t = jnp.tanh(u)
        d = t[:, :k] - t[:, k:]
        a = 0.5 * stanh(d) + 0.5
        return [hrs[i] + a[:, i:i + 1] * (hls[i] - hrs[i]) for i in range(k)]

    hs = [None] * NUM_NODES
    for n in range(NUM_NODES - 1, 6, -1):
        hs[n] = gru_leaf(x_ref[n])
    for lo, hi in ((3, 7), (1, 3), (0, 1)):
        parents = list(range(lo, hi))
        hsums = level_attn([hs[2 * n + 1] for n in parents],
                           [hs[2 * n + 2] for n in parents])
        for n, hsum in zip(parents, hsums):
            hs[n] = gru_internal(x_ref[n], hsum)

    m = hs[0]
    for n in range(1, NUM_NODES):
        m = jnp.maximum(m, hs[n])
    out_ref[...] = m


def _tree_encode(x, W_ih, W_hh, b_ih, b_hh, sent_weight, sent_bias, cw_row,
                 block_b: int):
    batch = x.shape[1]
    nb = batch // block_b
    b_leaf = (b_ih + jnp.concatenate(
        [b_hh[:, :2 * ENC], jnp.zeros((1, ENC), jnp.float32)], axis=1))
    b_rz = b_ih[:, :2 * ENC] + b_hh[:, :2 * ENC]
    b_in = b_ih[:, 2 * ENC:]
    b_hn = b_hh[:, 2 * ENC:]
    full = lambda shape: pl.BlockSpec(shape, lambda i: (0,) * len(shape))
    return pl.pallas_call(
        _tree_body,
        grid=(nb,),
        in_specs=[
            pl.BlockSpec((NUM_NODES, block_b, EMB), lambda i: (0, i, 0)),
            full((3 * ENC, EMB)),
            full((3 * ENC, ENC)),
            full((1, 3 * ENC)),
            full((1, 2 * ENC)),
            full((1, ENC)),
            full((1, ENC)),
            full((ENC, ENC)),
            full((1, ENC)),
            full((1, ENC)),
        ],
        out_specs=pl.BlockSpec((block_b, ENC), lambda i: (i, 0)),
        out_shape=jax.ShapeDtypeStruct((batch, ENC), jnp.float32),
        compiler_params=pltpu.CompilerParams(
            dimension_semantics=("parallel",)),
    )(x, W_ih, W_hh, b_leaf, b_rz, b_in, b_hn, sent_weight, sent_bias, cw_row)


NUM_SLICES = 4


def kernel(tokens, bs, emb_table, sent_weight, sent_bias, context_weight,
           W_ih, W_hh, b_ih, b_hh):
    num_nodes, batch = tokens.shape
    bsl = batch // NUM_SLICES
    gather = _make_sc_gather(emb_table.shape[0], num_nodes * bsl)
    outs = []
    for s in range(NUM_SLICES):
        idx = tokens[:, s * bsl:(s + 1) * bsl].reshape(-1).astype(jnp.int32)
        gath = gather(emb_table, idx)
        x = gath.reshape(num_nodes, bsl, EMB)
        outs.append(_tree_encode(
            x, W_ih, W_hh,
            b_ih.reshape(1, -1), b_hh.reshape(1, -1),
            sent_weight, sent_bias, context_weight.reshape(1, -1),
            block_b=512))
    return jnp.concatenate(outs, axis=0)


# speedup vs baseline: 1.0654x; 1.0380x over previous
import functools

import jax
import jax.numpy as jnp
from jax import lax
from jax.experimental import pallas as pl
from jax.experimental.pallas import tpu as pltpu
from jax.experimental.pallas import tpu_sc as plsc

EMB = 128
ENC = 128
NUM_NODES = 15
CHUNK = 128


@functools.lru_cache(maxsize=None)
def _make_sc_gather(vocab: int, rows_total: int):
    info = plsc.get_sparse_core_info()
    nw = info.num_cores * info.num_subcores
    assert rows_total % (nw * CHUNK) == 0
    rows_per_w = rows_total // nw
    n_chunks = rows_per_w // CHUNK
    mesh = plsc.VectorSubcoreMesh(core_axis_name="c", subcore_axis_name="s")

    @functools.partial(
        pl.kernel,
        out_type=jax.ShapeDtypeStruct((rows_total, EMB), jnp.float32),
        mesh=mesh,
        scratch_types=[
            pltpu.VMEM((rows_per_w,), jnp.int32),
            pltpu.VMEM((2, CHUNK, EMB), jnp.float32),
            pltpu.SemaphoreType.DMA,
            pltpu.SemaphoreType.DMA,
        ],
    )
    def gather_kernel(table_hbm, idx_hbm, out_hbm, idx_v, rows_v, gsem, osem):
        wid = lax.axis_index("s") * info.num_cores + lax.axis_index("c")
        base = wid * rows_per_w
        pltpu.sync_copy(idx_hbm.at[pl.ds(base, rows_per_w)], idx_v)

        def gather_start(t, buf):
            return pltpu.async_copy(
                table_hbm.at[idx_v.at[pl.ds(t * CHUNK, CHUNK)]],
                rows_v.at[buf], gsem)

        gather_start(0, 0).wait()
        def step(t, _):
            buf = lax.rem(t, 2)

            @pl.when(t + 1 < n_chunks)
            def _():
                gather_start(t + 1, 1 - buf)

            cp = pltpu.async_copy(
                rows_v.at[buf], out_hbm.at[pl.ds(base + t * CHUNK, CHUNK)],
                osem)

            @pl.when(t + 1 < n_chunks)
            def _():
                pltpu.make_async_copy(
                    table_hbm.at[idx_v.at[pl.ds(0, CHUNK)]],
                    rows_v.at[1 - buf], gsem).wait()
            cp.wait()
            return 0

        lax.fori_loop(0, n_chunks, step, 0)

    return gather_kernel


def _tree_body(x_ref, wih_ref, whh_ref, bleaf_ref, brz_ref, bin_ref, bhn_ref,
               sw_ref, sb_ref, cw_ref, out_ref):
    bf16 = jnp.bfloat16
    wih = wih_ref[...].astype(bf16)
    whh = whh_ref[...].astype(bf16)
    bleaf = bleaf_ref[...]
    brz = brz_ref[...]
    b_in = bin_ref[...]
    b_hn = bhn_ref[...]
    sw = sw_ref[...].astype(bf16)
    sb = sb_ref[...]
    cw = cw_ref[...]

    def mm_t(a, b):
        return lax.dot_general(a.astype(bf16), b, (((1,), (1,)), ((), ())),
                               preferred_element_type=jnp.float32)

    def mm(a, b):
        return lax.dot_general(a.astype(bf16), b, (((1,), (0,)), ((), ())),
                               preferred_element_type=jnp.float32)

    def stanh(x):
        return jnp.tanh(0.5 * x)

    def gru_leaf(xn):
        gi = mm_t(xn, wih) + bleaf
        tr = stanh(gi[:, :ENC])
        tz = stanh(gi[:, ENC:2 * ENC])
        n = jnp.tanh(gi[:, 2 * ENC:] + 0.5 * (1.0 + tr) * b_hn)
        return 0.5 * (n - tz * n)

    def gru_internal(xn, h):
        gi = mm_t(xn, wih)
        gh = mm_t(h, whh)
        grz = gi[:, :2 * ENC] + gh[:, :2 * ENC] + brz
        tr = stanh(grz[:, :ENC])
        tz = stanh(grz[:, ENC:])
        h_n = gh[:, 2 * ENC:] + b_hn
        n = jnp.tanh(gi[:, 2 * ENC:] + b_in + 0.5 * (h_n + tr * h_n))
        return 0.5 * (n + h + tz * (h - n))

    def level_attn(hls, hrs):
        def u_of(h):
            w1 = jnp.tanh(mm(h, sw) + sb)
            return jnp.sum(w1 * cw, axis=1, keepdims=True)
        k = len(hls)
        u = jnp.concatenate([u_of(h) for h in hls + hrs], axis=1)
        t = jnp.tanh(u)
        d = t[:, :k] - t[:, k:]
        a = 0.5 * stanh(d) + 0.5
        return [hrs[i] + a[:, i:i + 1] * (hls[i] - hrs[i]) for i in range(k)]

    hs = [None] * NUM_NODES
    for n in range(NUM_NODES - 1, 6, -1):
        hs[n] = gru_leaf(x_ref[n])
    for lo, hi in ((3, 7), (1, 3), (0, 1)):
        parents = list(range(lo, hi))
        hsums = level_attn([hs[2 * n + 1] for n in parents],
                           [hs[2 * n + 2] for n in parents])
        for n, hsum in zip(parents, hsums):
            hs[n] = gru_internal(x_ref[n], hsum)

    m = hs[0]
    for n in range(1, NUM_NODES):
        m = jnp.maximum(m, hs[n])
    out_ref[...] = m


def _tree_encode(x, W_ih, W_hh, b_ih, b_hh, sent_weight, sent_bias, cw_row,
                 block_b: int):
    batch = x.shape[1]
    nb = batch // block_b
    b_leaf = (b_ih + jnp.concatenate(
        [b_hh[:, :2 * ENC], jnp.zeros((1, ENC), jnp.float32)], axis=1))
    b_rz = b_ih[:, :2 * ENC] + b_hh[:, :2 * ENC]
    b_in = b_ih[:, 2 * ENC:]
    b_hn = b_hh[:, 2 * ENC:]
    full = lambda shape: pl.BlockSpec(shape, lambda i: (0,) * len(shape))
    return pl.pallas_call(
        _tree_body,
        grid=(nb,),
        in_specs=[
            pl.BlockSpec((NUM_NODES, block_b, EMB), lambda i: (0, i, 0)),
            full((3 * ENC, EMB)),
            full((3 * ENC, ENC)),
            full((1, 3 * ENC)),
            full((1, 2 * ENC)),
            full((1, ENC)),
            full((1, ENC)),
            full((ENC, ENC)),
            full((1, ENC)),
            full((1, ENC)),
        ],
        out_specs=pl.BlockSpec((block_b, ENC), lambda i: (i, 0)),
        out_shape=jax.ShapeDtypeStruct((batch, ENC), jnp.float32),
        compiler_params=pltpu.CompilerParams(
            dimension_semantics=("parallel",)),
    )(x, W_ih, W_hh, b_leaf, b_rz, b_in, b_hn, sent_weight, sent_bias, cw_row)


NUM_SLICES = 4


def kernel(tokens, bs, emb_table, sent_weight, sent_bias, context_weight,
           W_ih, W_hh, b_ih, b_hh):
    num_nodes, batch = tokens.shape
    bsl = batch // NUM_SLICES
    gather = _make_sc_gather(emb_table.shape[0], num_nodes * bsl)
    outs = []
    for s in range(NUM_SLICES):
        idx = tokens[:, s * bsl:(s + 1) * bsl].reshape(-1).astype(jnp.int32)
        gath = gather(emb_table, idx)
        x = gath.reshape(num_nodes, bsl, EMB)
        outs.append(_tree_encode(
            x, W_ih, W_hh,
            b_ih.reshape(1, -1), b_hh.reshape(1, -1),
            sent_weight, sent_bias, context_weight.reshape(1, -1),
            block_b=512))
    return jnp.concatenate(outs, axis=0)


# speedup vs baseline: 1.1148x; 1.0463x over previous
import functools

import jax
import jax.numpy as jnp
from jax import lax
from jax.experimental import pallas as pl
from jax.experimental.pallas import tpu as pltpu
from jax.experimental.pallas import tpu_sc as plsc

EMB = 128
ENC = 128
NUM_NODES = 15
@functools.lru_cache(maxsize=None)
def _make_sc_gather(vocab: int, rows_total: int):
    info = plsc.get_sparse_core_info()
    nw = info.num_cores * info.num_subcores
    rows_per_w = rows_total // nw
    CHUNK = next(c for c in (128, 96, 64, 32) if rows_per_w % c == 0)
    n_chunks = rows_per_w // CHUNK
    mesh = plsc.VectorSubcoreMesh(core_axis_name="c", subcore_axis_name="s")

    @functools.partial(
        pl.kernel,
        out_type=jax.ShapeDtypeStruct((rows_total, EMB), jnp.float32),
        mesh=mesh,
        scratch_types=[
            pltpu.VMEM((rows_per_w,), jnp.int32),
            pltpu.VMEM((2, CHUNK, EMB), jnp.float32),
            pltpu.SemaphoreType.DMA,
            pltpu.SemaphoreType.DMA,
        ],
    )
    def gather_kernel(table_hbm, idx_hbm, out_hbm, idx_v, rows_v, gsem, osem):
        wid = lax.axis_index("s") * info.num_cores + lax.axis_index("c")
        base = wid * rows_per_w
        pltpu.sync_copy(idx_hbm.at[pl.ds(base, rows_per_w)], idx_v)

        def gather_start(t, buf):
            return pltpu.async_copy(
                table_hbm.at[idx_v.at[pl.ds(t * CHUNK, CHUNK)]],
                rows_v.at[buf], gsem)

        gather_start(0, 0).wait()
        def step(t, _):
            buf = lax.rem(t, 2)

            @pl.when(t + 1 < n_chunks)
            def _():
                gather_start(t + 1, 1 - buf)

            cp = pltpu.async_copy(
                rows_v.at[buf], out_hbm.at[pl.ds(base + t * CHUNK, CHUNK)],
                osem)

            @pl.when(t + 1 < n_chunks)
            def _():
                pltpu.make_async_copy(
                    table_hbm.at[idx_v.at[pl.ds(0, CHUNK)]],
                    rows_v.at[1 - buf], gsem).wait()
            cp.wait()
            return 0

        lax.fori_loop(0, n_chunks, step, 0)

    return gather_kernel


def _tree_body(x_ref, wih_ref, whh_ref, bleaf_ref, brz_ref, bin_ref, bhn_ref,
               sw_ref, sb_ref, cw_ref, out_ref):
    bf16 = jnp.bfloat16
    wih = wih_ref[...].astype(bf16)
    whh = whh_ref[...].astype(bf16)
    bleaf = bleaf_ref[...]
    brz = brz_ref[...]
    b_in = bin_ref[...]
    b_hn = bhn_ref[...]
    sw = sw_ref[...].astype(bf16)
    sb = sb_ref[...]
    cw = cw_ref[...]

    def mm_t(a, b):
        return lax.dot_general(a.astype(bf16), b, (((1,), (1,)), ((), ())),
                               preferred_element_type=jnp.float32)

    def mm(a, b):
        return lax.dot_general(a.astype(bf16), b, (((1,), (0,)), ((), ())),
                               preferred_element_type=jnp.float32)

    def gru_leaf(xn):
        gi = mm_t(xn, wih) + bleaf
        tr = jnp.tanh(gi[:, :ENC])
        tz = jnp.tanh(gi[:, ENC:2 * ENC])
        n = jnp.tanh(gi[:, 2 * ENC:] + b_hn + b_hn * tr)
        return 0.5 * (n - tz * n)

    def gru_internal(xn, h):
        gi = mm_t(xn, wih)
        gh = mm_t(h, whh)
        grz = gi[:, :2 * ENC] + gh[:, :2 * ENC] + brz
        tr = jnp.tanh(grz[:, :ENC])
        tz = jnp.tanh(grz[:, ENC:])
        h_n2 = gh[:, 2 * ENC:] + b_hn
        n = jnp.tanh(gi[:, 2 * ENC:] + b_in + h_n2 + tr * h_n2)
        return 0.5 * (n + h + tz * (h - n))

    def level_attn(hls, hrs):
        def u_of(h):
            w1 = jnp.tanh(mm(h, sw) + sb)
            return jnp.sum(w1 * cw, axis=1, keepdims=True)
        k = len(hls)
        u = jnp.concatenate([u_of(h) for h in hls + hrs], axis=1)
        t = jnp.tanh(u)
        d = t[:, :k] - t[:, k:]
        a = 0.5 * jnp.tanh(0.5 * d) + 0.5
        return [hrs[i] + a[:, i:i + 1] * (hls[i] - hrs[i]) for i in range(k)]

    hs = [None] * NUM_NODES
    for n in range(NUM_NODES - 1, 6, -1):
        hs[n] = gru_leaf(x_ref[n])
    for lo, hi in ((3, 7), (1, 3), (0, 1)):
        parents = list(range(lo, hi))
        hsums = level_attn([hs[2 * n + 1] for n in parents],
                           [hs[2 * n + 2] for n in parents])
        for n, hsum in zip(parents, hsums):
            hs[n] = gru_internal(x_ref[n], hsum)

    m = hs[0]
    for n in range(1, NUM_NODES):
        m = jnp.maximum(m, hs[n])
    out_ref[...] = m


def _tree_encode(x, W_ih, W_hh, b_ih, b_hh, sent_weight, sent_bias, cw_row,
                 block_b: int):
    batch = x.shape[1]
    nb = batch // block_b
    W_ih = jnp.concatenate([0.5 * W_ih[:2 * ENC], W_ih[2 * ENC:]], axis=0)
    W_hh = 0.5 * W_hh
    b_leaf = jnp.concatenate(
        [0.5 * (b_ih[:, :2 * ENC] + b_hh[:, :2 * ENC]), b_ih[:, 2 * ENC:]],
        axis=1)
    b_rz = 0.5 * (b_ih[:, :2 * ENC] + b_hh[:, :2 * ENC])
    b_in = b_ih[:, 2 * ENC:]
    b_hn = 0.5 * b_hh[:, 2 * ENC:]
    full = lambda shape: pl.BlockSpec(shape, lambda i: (0,) * len(shape))
    return pl.pallas_call(
        _tree_body,
        grid=(nb,),
        in_specs=[
            pl.BlockSpec((NUM_NODES, block_b, EMB), lambda i: (0, i, 0)),
            full((3 * ENC, EMB)),
            full((3 * ENC, ENC)),
            full((1, 3 * ENC)),
            full((1, 2 * ENC)),
            full((1, ENC)),
            full((1, ENC)),
            full((ENC, ENC)),
            full((1, ENC)),
            full((1, ENC)),
        ],
        out_specs=pl.BlockSpec((block_b, ENC), lambda i: (i, 0)),
        out_shape=jax.ShapeDtypeStruct((batch, ENC), jnp.float32),
        compiler_params=pltpu.CompilerParams(
            dimension_semantics=("parallel",)),
    )(x, W_ih, W_hh, b_leaf, b_rz, b_in, b_hn, sent_weight, sent_bias, cw_row)


NUM_SLICES = 8


def kernel(tokens, bs, emb_table, sent_weight, sent_bias, context_weight,
           W_ih, W_hh, b_ih, b_hh):
    num_nodes, batch = tokens.shape
    bsl = batch // NUM_SLICES
    gather = _make_sc_gather(emb_table.shape[0], num_nodes * bsl)
    outs = []
    for s in range(NUM_SLICES):
        idx = tokens[:, s * bsl:(s + 1) * bsl].reshape(-1).astype(jnp.int32)
        gath = gather(emb_table, idx)
        x = gath.reshape(num_nodes, bsl, EMB)
        outs.append(_tree_encode(
            x, W_ih, W_hh,
            b_ih.reshape(1, -1), b_hh.reshape(1, -1),
            sent_weight, sent_bias, context_weight.reshape(1, -1),
            block_b=512))
    return jnp.concatenate(outs, axis=0)


# speedup vs baseline: 1.1480x; 1.0298x over previous
import functools

import jax
import jax.numpy as jnp
from jax import lax
from jax.experimental import pallas as pl
from jax.experimental.pallas import tpu as pltpu
from jax.experimental.pallas import tpu_sc as plsc

EMB = 128
ENC = 128
NUM_NODES = 15
@functools.lru_cache(maxsize=None)
def _make_sc_gather(vocab: int, rows_total: int):
    info = plsc.get_sparse_core_info()
    nw = info.num_cores * info.num_subcores
    rows_per_w = rows_total // nw
    CHUNK = next(c for c in (128, 96, 64, 32) if rows_per_w % c == 0)
    n_chunks = rows_per_w // CHUNK
    mesh = plsc.VectorSubcoreMesh(core_axis_name="c", subcore_axis_name="s")

    @functools.partial(
        pl.kernel,
        out_type=jax.ShapeDtypeStruct((rows_total, EMB), jnp.float32),
        mesh=mesh,
        scratch_types=[
            pltpu.VMEM((rows_per_w,), jnp.int32),
            pltpu.VMEM((2, CHUNK, EMB), jnp.float32),
            pltpu.SemaphoreType.DMA,
            pltpu.SemaphoreType.DMA,
        ],
    )
    def gather_kernel(table_hbm, idx_hbm, out_hbm, idx_v, rows_v, gsem, osem):
        wid = lax.axis_index("s") * info.num_cores + lax.axis_index("c")
        base = wid * rows_per_w
        pltpu.sync_copy(idx_hbm.at[pl.ds(base, rows_per_w)], idx_v)

        def gather_start(t, buf):
            return pltpu.async_copy(
                table_hbm.at[idx_v.at[pl.ds(t * CHUNK, CHUNK)]],
                rows_v.at[buf], gsem)

        gather_start(0, 0).wait()
        def step(t, _):
            buf = lax.rem(t, 2)

            @pl.when(t + 1 < n_chunks)
            def _():
                gather_start(t + 1, 1 - buf)

            cp = pltpu.async_copy(
                rows_v.at[buf], out_hbm.at[pl.ds(base + t * CHUNK, CHUNK)],
                osem)

            @pl.when(t + 1 < n_chunks)
            def _():
                pltpu.make_async_copy(
                    table_hbm.at[idx_v.at[pl.ds(0, CHUNK)]],
                    rows_v.at[1 - buf], gsem).wait()
            cp.wait()
            return 0

        lax.fori_loop(0, n_chunks, step, 0)

    return gather_kernel


def _tree_body(x_ref, wih_ref, whh_ref, bleaf_ref, brz_ref, bin_ref, bhn_ref,
               sw_ref, sb_ref, cw_ref, buf_ref, out_ref):
    del buf_ref
    bf16 = jnp.bfloat16
    wih = wih_ref[...].astype(bf16)
    whh = whh_ref[...].astype(bf16)
    bleaf = bleaf_ref[...]
    brz = brz_ref[...]
    b_in = bin_ref[...]
    b_hn = bhn_ref[...]
    sw = sw_ref[...].astype(bf16)
    sb = sb_ref[...]
    cw = cw_ref[...]

    def mm_t(a, b):
        return lax.dot_general(a.astype(bf16), b, (((1,), (1,)), ((), ())),
                               preferred_element_type=jnp.float32)

    def mm(a, b):
        return lax.dot_general(a.astype(bf16), b, (((1,), (0,)), ((), ())),
                               preferred_element_type=jnp.float32)

    def gru_leaf(xn):
        gi = mm_t(xn, wih) + bleaf
        tr = jnp.tanh(gi[:, :ENC])
        tz = jnp.tanh(gi[:, ENC:2 * ENC])
        n = jnp.tanh(gi[:, 2 * ENC:] + b_hn + b_hn * tr)
        return 0.5 * (n - tz * n)

    def gru_internal(xn, h):
        gi = mm_t(xn, wih)
        gh = mm_t(h, whh)
        grz = gi[:, :2 * ENC] + gh[:, :2 * ENC] + brz
        tr = jnp.tanh(grz[:, :ENC])
        tz = jnp.tanh(grz[:, ENC:])
        h_n2 = gh[:, 2 * ENC:] + b_hn
        n = jnp.tanh(gi[:, 2 * ENC:] + b_in + h_n2 + tr * h_n2)
        return 0.5 * (n + h + tz * (h - n))

    def level_attn(hls, hrs):
        def u_of(h):
            w1 = jnp.tanh(mm(h, sw) + sb)
            return jnp.sum(w1 * cw, axis=1, keepdims=True)
        k = len(hls)
        u = jnp.concatenate([u_of(h) for h in hls + hrs], axis=1)
        t = jnp.tanh(u)
        d = t[:, :k] - t[:, k:]
        a = 0.5 * jnp.tanh(0.5 * d) + 0.5
        return [hrs[i] + a[:, i:i + 1] * (hls[i] - hrs[i]) for i in range(k)]

    hs = [None] * NUM_NODES
    for n in range(NUM_NODES - 1, 6, -1):
        hs[n] = gru_leaf(x_ref[n])
    for lo, hi in ((3, 7), (1, 3), (0, 1)):
        parents = list(range(lo, hi))
        hsums = level_attn([hs[2 * n + 1] for n in parents],
                           [hs[2 * n + 2] for n in parents])
        for n, hsum in zip(parents, hsums):
            hs[n] = gru_internal(x_ref[n], hsum)

    m = hs[0]
    for n in range(1, NUM_NODES):
        m = jnp.maximum(m, hs[n])
    out_ref[...] = m


def _tree_encode(x, W_ih, W_hh, b_ih, b_hh, sent_weight, sent_bias, cw_row,
                 block_b: int, out_buf, block_base: int):
    batch = x.shape[1]
    nb = batch // block_b
    W_ih = jnp.concatenate([0.5 * W_ih[:2 * ENC], W_ih[2 * ENC:]], axis=0)
    W_hh = 0.5 * W_hh
    b_leaf = jnp.concatenate(
        [0.5 * (b_ih[:, :2 * ENC] + b_hh[:, :2 * ENC]), b_ih[:, 2 * ENC:]],
        axis=1)
    b_rz = 0.5 * (b_ih[:, :2 * ENC] + b_hh[:, :2 * ENC])
    b_in = b_ih[:, 2 * ENC:]
    b_hn = 0.5 * b_hh[:, 2 * ENC:]
    full = lambda shape: pl.BlockSpec(shape, lambda i: (0,) * len(shape))
    return pl.pallas_call(
        _tree_body,
        grid=(nb,),
        in_specs=[
            pl.BlockSpec((NUM_NODES, block_b, EMB), lambda i: (0, i, 0)),
            full((3 * ENC, EMB)),
            full((3 * ENC, ENC)),
            full((1, 3 * ENC)),
            full((1, 2 * ENC)),
            full((1, ENC)),
            full((1, ENC)),
            full((ENC, ENC)),
            full((1, ENC)),
            full((1, ENC)),
            pl.BlockSpec(memory_space=pltpu.MemorySpace.HBM),
        ],
        out_specs=pl.BlockSpec((block_b, ENC), lambda i: (block_base + i, 0)),
        out_shape=jax.ShapeDtypeStruct(out_buf.shape, jnp.float32),
        input_output_aliases={10: 0},
        compiler_params=pltpu.CompilerParams(
            dimension_semantics=("parallel",)),
    )(x, W_ih, W_hh, b_leaf, b_rz, b_in, b_hn, sent_weight, sent_bias, cw_row,
      out_buf)


NUM_SLICES = 8


def kernel(tokens, bs, emb_table, sent_weight, sent_bias, context_weight,
           W_ih, W_hh, b_ih, b_hh):
    num_nodes, batch = tokens.shape
    bsl = batch // NUM_SLICES
    block_b = 512
    gather = _make_sc_gather(emb_table.shape[0], num_nodes * bsl)
    out = jnp.zeros((batch, ENC), jnp.float32)
    for s in range(NUM_SLICES):
        idx = tokens[:, s * bsl:(s + 1) * bsl].reshape(-1).astype(jnp.int32)
        gath = gather(emb_table, idx)
        x = gath.reshape(num_nodes, bsl, EMB)
        out = _tree_encode(
            x, W_ih, W_hh,
            b_ih.reshape(1, -1), b_hh.reshape(1, -1),
            sent_weight, sent_bias, context_weight.reshape(1, -1),
            block_b=block_b, out_buf=out,
            block_base=s * (bsl // block_b))
    return out
